# baseline (speedup 1.0000x reference)
import jax
import jax.numpy as jnp
from jax import lax
from jax.experimental import pallas as pl
from jax.experimental.pallas import tpu as pltpu
from jax.experimental.pallas import tpu_sc as plsc

_B = 4096
_F = 26
_V = 100000
_D = 32
_NC = 16

_NW = 32
_BPW = _B // _NW
_F2 = 2 * _F
_VBLK = 102400
_NVB = (_V + _VBLK - 1) // _VBLK
_VPAD = _NVB * _VBLK


def _proj_body(w_ref, e_ref, s_ref):
    for i in range(2):
        x = e_ref[i]
        w = w_ref[i]
        s_ref[pl.ds(i * _VBLK, _VBLK)] = jnp.sum(x * w, axis=0)


def _project(embT, wT):
    return pl.pallas_call(
        _proj_body,
        grid=(_F // 2, _NVB),
        in_specs=[
            pl.BlockSpec((2, _D, 1), lambda f, vb: (f, 0, 0)),
            pl.BlockSpec((2, _D, _VBLK), lambda f, vb: (f, 0, vb)),
        ],
        out_specs=pl.BlockSpec((2 * _VBLK,), lambda f, vb: (f * _NVB + vb,)),
        out_shape=jax.ShapeDtypeStruct((_F * _VPAD,), jnp.float32),
        compiler_params=pltpu.CompilerParams(
            vmem_limit_bytes=100 * 1024 * 1024),
    )(wT, embT)


def _sc_body(s1, cat2, num2, kdup, foff, out,
             catv, numv, kv, foffv, idxv, sv, outv, sem):
    wid = lax.axis_index("s") * 2 + lax.axis_index("c")
    b0 = wid * _BPW
    iota = lax.iota(jnp.int32, 16)
    zero16 = jnp.zeros((16,), jnp.float32)

    pltpu.sync_copy(cat2.at[pl.ds(b0, _BPW), :], catv)
    pltpu.sync_copy(num2.at[pl.ds(b0, _BPW), :], numv)
    pltpu.sync_copy(kdup, kv)
    pltpu.sync_copy(foff, foffv)

    def build(fp, _):
        fpv = jnp.full((16,), fp, dtype=jnp.int32)
        fofb = plsc.load_gather(foffv, [fpv])
        for g in range(8):
            jvec = iota + (g * 16)
            catg = plsc.load_gather(catv, [jvec, fpv])
            idxv[fp, pl.ds(g * 16, 16)] = catg + fofb
        return 0

    lax.fori_loop(0, _F2, build, 0)

    descs = [
        pltpu.async_copy(s1.at[idxv.at[f]], sv.at[f], sem)
        for f in range(_F2)
    ]
    for d in descs:
        d.wait()

    for g in range(8):
        jvec = iota + (g * 16)

        def body_add(fp, a):
            return a + sv[fp, pl.ds(g * 16, 16)]

        accp = lax.fori_loop(0, _F, body_add, zero16)
        accn = lax.fori_loop(_F, _F2, body_add, zero16)
        acc = accp - accn

        def body_num(l, a):
            kb = plsc.load_gather(kv, [jnp.full((16,), l, dtype=jnp.int32)])
            nv = plsc.load_gather(numv, [jvec,
                                         jnp.full((16,), l, dtype=jnp.int32)])
            return a + kb * nv

        acc = lax.fori_loop(0, 2 * _NC, body_num, acc)
        outv[pl.ds(g * 16, 16)] = acc

    pltpu.sync_copy(outv, out.at[pl.ds(b0, _BPW)])


def _gather_reduce(s1, cat2, num2, kdup, foff):
    mesh = plsc.VectorSubcoreMesh(core_axis_name="c", subcore_axis_name="s",
                                  num_cores=2, num_subcores=16)
    fn = pl.kernel(
        _sc_body,
        out_type=jax.ShapeDtypeStruct((_B,), jnp.float32),
        mesh=mesh,
        scratch_types=[
            pltpu.VMEM((_BPW, _F2), jnp.int32),
            pltpu.VMEM((_BPW, 2 * _NC), jnp.float32),
            pltpu.VMEM((2 * _NC,), jnp.float32),
            pltpu.VMEM((_F2,), jnp.int32),
            pltpu.VMEM((_F2, _BPW), jnp.int32),
            pltpu.VMEM((_F2, _BPW), jnp.float32),
            pltpu.VMEM((_BPW,), jnp.float32),
            pltpu.SemaphoreType.DMA,
        ],
        compiler_params=pltpu.CompilerParams(needs_layout_passes=False,
                                             use_tc_tiling_on_sc=False),
    )
    return fn(s1, cat2, num2, kdup, foff)


@jax.jit
def _run(embT, wT, cat2, num2, kdup, foff):
    s = _project(embT, wT)
    return _gather_reduce(s, cat2, num2, kdup, foff)


def kernel(pos_cat, pos_num, neg_cat, neg_num, emb_tables, lin_w, lin_b,
           bn_gamma, bn_beta, bn_mean, bn_var):
    embT = jnp.transpose(emb_tables, (0, 2, 1))
    w_emb = lin_w[0, : _F * _D].reshape(_F, _D)
    wT = w_emb.reshape(_F, _D, 1)
    cat2 = jnp.concatenate([pos_cat, neg_cat], axis=1)
    num2 = jnp.concatenate([pos_num, neg_num], axis=1)
    knum = lin_w[0, _F * _D:] * bn_gamma * lax.rsqrt(bn_var + 1e-5)
    kdup = jnp.concatenate([knum, -knum], axis=0)
    foff = jnp.tile(jnp.arange(_F, dtype=jnp.int32) * _VPAD, 2)
    out = _run(embT, wT, cat2, num2, kdup, foff)
    return out.reshape(_B, 1)


# speedup vs baseline: 1.0394x; 1.0394x over previous
import jax
import jax.numpy as jnp
from jax import lax
from jax.experimental import pallas as pl
from jax.experimental.pallas import tpu as pltpu
from jax.experimental.pallas import tpu_sc as plsc

_B = 4096
_F = 26
_V = 100000
_D = 32
_NC = 16

_NW = 32
_BPW = _B // _NW
_F2 = 2 * _F
_VBLK = 102400
_NVB = (_V + _VBLK - 1) // _VBLK
_VPAD = _NVB * _VBLK


def _proj_body(w_ref, e_ref, s_ref):
    x = e_ref[0]
    w = w_ref[0]
    s_ref[...] = jnp.sum(x * w, axis=0)


def _project(embT, wT):
    return pl.pallas_call(
        _proj_body,
        grid=(_F, _NVB),
        in_specs=[
            pl.BlockSpec((1, _D, 1), lambda f, vb: (f, 0, 0)),
            pl.BlockSpec((1, _D, _VBLK), lambda f, vb: (f, 0, vb)),
        ],
        out_specs=pl.BlockSpec((_VBLK,), lambda f, vb: (f * _NVB + vb,)),
        out_shape=jax.ShapeDtypeStruct((_F * _VPAD,), jnp.float32),
        compiler_params=pltpu.CompilerParams(
            vmem_limit_bytes=100 * 1024 * 1024),
    )(wT, embT)


def _sc_body(s1, cat2, num2, kdup, foff, out,
             catv, numv, kv, foffv, idxv, sv, outv, sem):
    wid = lax.axis_index("s") * 2 + lax.axis_index("c")
    b0 = wid * _BPW
    iota = lax.iota(jnp.int32, 16)
    zero16 = jnp.zeros((16,), jnp.float32)

    stage = [
        pltpu.async_copy(cat2.at[pl.ds(b0, _BPW), :], catv, sem),
        pltpu.async_copy(num2.at[pl.ds(b0, _BPW), :], numv, sem),
        pltpu.async_copy(kdup, kv, sem),
        pltpu.async_copy(foff, foffv, sem),
    ]
    for d in stage:
        d.wait()

    def build(fp, _):
        fpv = jnp.full((16,), fp, dtype=jnp.int32)
        fofb = plsc.load_gather(foffv, [fpv])
        for g in range(8):
            jvec = iota + (g * 16)
            catg = plsc.load_gather(catv, [jvec, fpv])
            idxv[fp, pl.ds(g * 16, 16)] = catg + fofb
        return 0

    lax.fori_loop(0, _F2, build, 0, unroll=4)

    descs = [
        pltpu.async_copy(s1.at[idxv.at[f]], sv.at[f], sem)
        for f in range(_F2)
    ]
    for d in descs:
        d.wait()

    for g in range(8):
        jvec = iota + (g * 16)

        def body_add(fp, a):
            return a + sv[fp, pl.ds(g * 16, 16)]

        accp = lax.fori_loop(0, _F, body_add, zero16, unroll=13)
        accn = lax.fori_loop(_F, _F2, body_add, zero16, unroll=13)
        acc = accp - accn

        def body_num(l, a):
            kb = plsc.load_gather(kv, [jnp.full((16,), l, dtype=jnp.int32)])
            nv = plsc.load_gather(numv, [jvec,
                                         jnp.full((16,), l, dtype=jnp.int32)])
            return a + kb * nv

        acc = lax.fori_loop(0, 2 * _NC, body_num, acc, unroll=8)
        outv[pl.ds(g * 16, 16)] = acc

    pltpu.sync_copy(outv, out.at[pl.ds(b0, _BPW)])


def _gather_reduce(s1, cat2, num2, kdup, foff):
    mesh = plsc.VectorSubcoreMesh(core_axis_name="c", subcore_axis_name="s",
                                  num_cores=2, num_subcores=16)
    fn = pl.kernel(
        _sc_body,
        out_type=jax.ShapeDtypeStruct((_B,), jnp.float32),
        mesh=mesh,
        scratch_types=[
            pltpu.VMEM((_BPW, _F2), jnp.int32),
            pltpu.VMEM((_BPW, 2 * _NC), jnp.float32),
            pltpu.VMEM((2 * _NC,), jnp.float32),
            pltpu.VMEM((_F2,), jnp.int32),
            pltpu.VMEM((_F2, _BPW), jnp.int32),
            pltpu.VMEM((_F2, _BPW), jnp.float32),
            pltpu.VMEM((_BPW,), jnp.float32),
            pltpu.SemaphoreType.DMA,
        ],
        compiler_params=pltpu.CompilerParams(needs_layout_passes=False,
                                             use_tc_tiling_on_sc=False),
    )
    return fn(s1, cat2, num2, kdup, foff)


@jax.jit
def _run(embT, wT, cat2, num2, kdup, foff):
    s = _project(embT, wT)
    return _gather_reduce(s, cat2, num2, kdup, foff)


def kernel(pos_cat, pos_num, neg_cat, neg_num, emb_tables, lin_w, lin_b,
           bn_gamma, bn_beta, bn_mean, bn_var):
    embT = jnp.transpose(emb_tables, (0, 2, 1))
    w_emb = lin_w[0, : _F * _D].reshape(_F, _D)
    wT = w_emb.reshape(_F, _D, 1)
    cat2 = jnp.concatenate([pos_cat, neg_cat], axis=1)
    num2 = jnp.concatenate([pos_num, neg_num], axis=1)
    knum = lin_w[0, _F * _D:] * bn_gamma * lax.rsqrt(bn_var + 1e-5)
    kdup = jnp.concatenate([knum, -knum], axis=0)
    foff = jnp.tile(jnp.arange(_F, dtype=jnp.int32) * _VPAD, 2)
    out = _run(embT, wT, cat2, num2, kdup, foff)
    return out.reshape(_B, 1)
